# baseline (speedup 1.0000x reference)
import functools

import jax
import jax.numpy as jnp
from jax import lax
from jax.experimental import pallas as pl
from jax.experimental.pallas import tpu as pltpu
from jax.experimental.pallas import tpu_sc as plsc

NC = 2
NS = 16
CHUNK = 64
IDXB = 80


def _sc_mesh():
    return plsc.VectorSubcoreMesh(
        core_axis_name="c", subcore_axis_name="s",
        num_cores=NC, num_subcores=NS)


def _make_sc_agg(n_dst, n_chunks):
    assert n_dst % (NS * 8) == 0
    rows_per_tile = n_dst // NS

    @functools.partial(
        pl.kernel,
        out_type=jax.ShapeDtypeStruct((NC, n_dst, 128), jnp.float32),
        mesh=_sc_mesh(),
        scratch_types=[
            pltpu.VMEM((IDXB, CHUNK), jnp.int32),
            pltpu.VMEM((IDXB, CHUNK), jnp.int32),
            pltpu.VMEM((CHUNK, 128), jnp.float32),
            pltpu.VMEM((CHUNK, 128), jnp.float32),
            pltpu.VMEM_SHARED((n_dst, 128), jnp.float32),
            pltpu.SemaphoreType.DMA,
            pltpu.SemaphoreType.DMA,
        ],
    )
    def agg(src_hbm, dst_hbm, table_hbm, zrows_hbm, ssum_hbm,
            idx_s, idx_d, rows0, rows1, s_acc, sem0, sem1):
        c = lax.axis_index("c")
        sid = lax.axis_index("s")
        r0 = pl.multiple_of(sid * rows_per_tile, 8)

        pltpu.sync_copy(zrows_hbm.at[pl.ds(r0, rows_per_tile)],
                        s_acc.at[pl.ds(r0, rows_per_tile)])
        plsc.subcore_barrier()

        def start_g(j, buf, sem):
            pltpu.async_copy(table_hbm.at[idx_s.at[j]], buf, sem)

        def wait_g(buf, sem):
            pltpu.make_async_copy(zrows_hbm.at[pl.ds(0, CHUNK)], buf,
                                  sem).wait()

        def scat(j, buf):
            pltpu.sync_copy(buf, s_acc.at[idx_d.at[j]], add=True)

        def outer(ob, carry):
            cb = pl.multiple_of(ob * IDXB, 8)
            pltpu.sync_copy(src_hbm.at[c, sid, pl.ds(cb, IDXB)], idx_s)
            pltpu.sync_copy(dst_hbm.at[c, sid, pl.ds(cb, IDXB)], idx_d)
            start_g(0, rows0, sem0)

            def pair(j2, cc):
                j0 = 2 * j2
                wait_g(rows0, sem0)
                start_g(j0 + 1, rows1, sem1)
                scat(j0, rows0)
                wait_g(rows1, sem1)

                @pl.when(j0 + 2 < IDXB)
                def _():
                    start_g(j0 + 2, rows0, sem0)

                scat(j0 + 1, rows1)
                return cc

            return lax.fori_loop(0, IDXB // 2, pair, carry, unroll=False)

        lax.fori_loop(0, n_chunks // IDXB, outer, 0, unroll=False)
        plsc.subcore_barrier()

        pltpu.sync_copy(s_acc.at[pl.ds(r0, rows_per_tile)],
                        ssum_hbm.at[c, pl.ds(r0, rows_per_tile)])

    return agg


def _make_sc_cnt(n_dst, n_chunks):
    assert n_dst % (NS * 8) == 0
    rows_per_tile = n_dst // NS

    @functools.partial(
        pl.kernel,
        out_type=jax.ShapeDtypeStruct((NC, n_dst, 128), jnp.float32),
        mesh=_sc_mesh(),
        scratch_types=[
            pltpu.VMEM((IDXB, CHUNK), jnp.int32),
            pltpu.VMEM((CHUNK, 128), jnp.float32),
            pltpu.VMEM_SHARED((n_dst, 128), jnp.float32),
        ],
    )
    def cntk(dst_hbm, zcnt_hbm, ones_hbm, cnt_hbm, idx_d, ones_v, c_acc):
        c = lax.axis_index("c")
        sid = lax.axis_index("s")
        r0 = pl.multiple_of(sid * rows_per_tile, 8)

        pltpu.sync_copy(zcnt_hbm.at[pl.ds(r0, rows_per_tile)],
                        c_acc.at[pl.ds(r0, rows_per_tile)])
        pltpu.sync_copy(ones_hbm, ones_v)
        plsc.subcore_barrier()

        def outer(ob, carry):
            cb = pl.multiple_of(ob * IDXB, 8)
            pltpu.sync_copy(dst_hbm.at[c, sid, pl.ds(cb, IDXB)], idx_d)

            def step(j, cc):
                pltpu.sync_copy(ones_v, c_acc.at[idx_d.at[j]], add=True)
                return cc

            return lax.fori_loop(0, IDXB, step, carry, unroll=False)

        lax.fori_loop(0, n_chunks // IDXB, outer, 0, unroll=False)
        plsc.subcore_barrier()

        pltpu.sync_copy(c_acc.at[pl.ds(r0, rows_per_tile)],
                        cnt_hbm.at[c, pl.ds(r0, rows_per_tile)])

    return cntk


def _tc_fused(ssum, cnt8, xc, Wagg, Wself, b, apply_relu):
    s_rows = ssum.shape[1]
    nb = 16
    blk = s_rows // nb
    assert blk % 8 == 0 and xc.shape[0] == 2 * s_rows

    def body(s_ref, c_ref, x_ref, wa_ref, ws_ref, b_ref, o_ref):
        cntcol = c_ref[0, :, 0:1]
        m = s_ref[0] / jnp.maximum(cntcol, 1.0)
        acc = jnp.dot(m, wa_ref[0], preferred_element_type=jnp.float32)
        acc = acc + jnp.dot(x_ref[...], ws_ref[0],
                            preferred_element_type=jnp.float32)
        acc = acc + b_ref[0]
        if apply_relu:
            acc = jnp.maximum(acc, 0.0)
        o_ref[...] = acc + x_ref[...]

    return pl.pallas_call(
        body,
        grid=(2, nb),
        in_specs=[
            pl.BlockSpec((1, blk, 128), lambda i, j: (i, j, 0)),
            pl.BlockSpec((1, blk, 128), lambda i, j: (i, j, 0)),
            pl.BlockSpec((blk, 128), lambda i, j: ((1 - i) * nb + j, 0)),
            pl.BlockSpec((1, 128, 128), lambda i, j: (i, 0, 0)),
            pl.BlockSpec((1, 128, 128), lambda i, j: (i, 0, 0)),
            pl.BlockSpec((1, 1, 128), lambda i, j: (i, 0, 0)),
        ],
        out_specs=pl.BlockSpec((blk, 128), lambda i, j: ((1 - i) * nb + j, 0)),
        out_shape=jax.ShapeDtypeStruct((2 * s_rows, 128), jnp.float32),
    )(ssum, cnt8, xc, Wagg, Wself, b.reshape(2, 1, 128))


def kernel(x_user, x_item, edge_index_u2i, edge_index_i2u, batch_user,
           batch_item, Wsu0, Wsi0, Wru0, Wri0, bu0, bi0,
           Wsu1, Wsi1, Wru1, Wri1, bu1, bi1):
    n_user, d = x_user.shape
    n_item = x_item.shape[0]
    e = edge_index_u2i.shape[1]
    assert n_user == n_item and d == 128
    n_dst = NS * 8 * (n_user // (NS * 8) + 1)

    n_chunks = IDXB * (-(-e // (NS * CHUNK * IDXB)))
    e_pad = NS * n_chunks * CHUNK
    pad = e_pad - e

    src0 = edge_index_u2i[0]
    dst0 = edge_index_u2i[1]
    src1 = edge_index_i2u[0] + n_dst
    dst1 = edge_index_i2u[1]
    src = jnp.stack([
        jnp.pad(src0, (0, pad)),
        jnp.pad(src1, (0, pad), constant_values=n_dst),
    ]).reshape(2, NS, n_chunks, CHUNK)
    dummy = n_dst - 1
    dst = jnp.stack([
        jnp.pad(dst0, (0, pad), constant_values=dummy),
        jnp.pad(dst1, (0, pad), constant_values=dummy),
    ]).reshape(2, NS, n_chunks, CHUNK)

    zrows = jnp.zeros((n_dst, 128), jnp.float32)
    ones = jnp.ones((CHUNK, 128), jnp.float32)

    agg = _make_sc_agg(n_dst, n_chunks)
    cntk = _make_sc_cnt(n_dst, n_chunks)

    Wagg0 = jnp.stack([Wru0, Wri0])
    Wself0 = jnp.stack([Wsi0, Wsu0])
    b0 = jnp.stack([bi0, bu0])
    Wagg1 = jnp.stack([Wru1, Wri1])
    Wself1 = jnp.stack([Wsi1, Wsu1])
    b1 = jnp.stack([bi1, bu1])

    cnt8 = cntk(dst, zrows, ones)

    zpad = jnp.zeros((n_dst - n_user, 128), jnp.float32)
    xc0 = jnp.concatenate([x_user, zpad, x_item, zpad], axis=0)
    ssum0 = agg(src, dst, xc0, zrows)
    xc1 = _tc_fused(ssum0, cnt8, xc0, Wagg0, Wself0, b0, apply_relu=True)

    ssum1 = agg(src, dst, xc1, zrows)
    xc2 = _tc_fused(ssum1, cnt8, xc1, Wagg1, Wself1, b1, apply_relu=False)

    return xc2[:n_user], xc2[n_dst:n_dst + n_user]


# speedup vs baseline: 1.1631x; 1.1631x over previous
import functools

import jax
import jax.numpy as jnp
from jax import lax
from jax.experimental import pallas as pl
from jax.experimental.pallas import tpu as pltpu
from jax.experimental.pallas import tpu_sc as plsc

NC = 2
NS = 16
CHUNK = 64
IDXB = 80


def _sc_mesh():
    return plsc.VectorSubcoreMesh(
        core_axis_name="c", subcore_axis_name="s",
        num_cores=NC, num_subcores=NS)


def _make_sc_agg(n_dst, n_chunks):
    assert n_dst % (NS * 8) == 0
    rows_per_tile = n_dst // NS

    @functools.partial(
        pl.kernel,
        out_type=jax.ShapeDtypeStruct((NC, n_dst, 128), jnp.float32),
        mesh=_sc_mesh(),
        scratch_types=[
            pltpu.VMEM((IDXB, CHUNK), jnp.int32),
            pltpu.VMEM((IDXB, CHUNK), jnp.int32),
            pltpu.VMEM((CHUNK, 128), jnp.float32),
            pltpu.VMEM((CHUNK, 128), jnp.float32),
            pltpu.VMEM_SHARED((n_dst, 128), jnp.float32),
            pltpu.SemaphoreType.DMA,
            pltpu.SemaphoreType.DMA,
        ],
    )
    def agg(src_hbm, dst_hbm, table_hbm, zrows_hbm, ssum_hbm,
            idx_s, idx_d, rows0, rows1, s_acc, sem0, sem1):
        c = lax.axis_index("c")
        sid = lax.axis_index("s")
        r0 = pl.multiple_of(sid * rows_per_tile, 8)

        pltpu.sync_copy(zrows_hbm.at[pl.ds(r0, rows_per_tile)],
                        s_acc.at[pl.ds(r0, rows_per_tile)])
        plsc.subcore_barrier()

        def start_g(j, buf, sem):
            pltpu.async_copy(table_hbm.at[idx_s.at[j]], buf, sem)

        def wait_g(buf, sem):
            pltpu.make_async_copy(zrows_hbm.at[pl.ds(0, CHUNK)], buf,
                                  sem).wait()

        def scat(j, buf):
            pltpu.sync_copy(buf, s_acc.at[idx_d.at[j]], add=True)

        def outer(ob, carry):
            cb = pl.multiple_of(ob * IDXB, 8)
            pltpu.sync_copy(src_hbm.at[c, sid, pl.ds(cb, IDXB)], idx_s)
            pltpu.sync_copy(dst_hbm.at[c, sid, pl.ds(cb, IDXB)], idx_d)
            start_g(0, rows0, sem0)

            def pair(j2, cc):
                j0 = 2 * j2
                wait_g(rows0, sem0)
                start_g(j0 + 1, rows1, sem1)
                scat(j0, rows0)
                wait_g(rows1, sem1)

                @pl.when(j0 + 2 < IDXB)
                def _():
                    start_g(j0 + 2, rows0, sem0)

                scat(j0 + 1, rows1)
                return cc

            return lax.fori_loop(0, IDXB // 2, pair, carry, unroll=False)

        lax.fori_loop(0, n_chunks // IDXB, outer, 0, unroll=False)
        plsc.subcore_barrier()

        pltpu.sync_copy(s_acc.at[pl.ds(r0, rows_per_tile)],
                        ssum_hbm.at[c, pl.ds(r0, rows_per_tile)])

    return agg


def _make_sc_cnt(n_dst, n_chunks):
    assert n_dst % (NS * 8) == 0
    rows_per_tile = n_dst // NS

    @functools.partial(
        pl.kernel,
        out_type=jax.ShapeDtypeStruct((NC, n_dst, 128), jnp.float32),
        mesh=_sc_mesh(),
        scratch_types=[
            pltpu.VMEM((IDXB, CHUNK), jnp.int32),
            pltpu.VMEM((CHUNK, 128), jnp.float32),
            pltpu.VMEM_SHARED((n_dst, 128), jnp.float32),
        ],
    )
    def cntk(dst_hbm, zcnt_hbm, ones_hbm, cnt_hbm, idx_d, ones_v, c_acc):
        c = lax.axis_index("c")
        sid = lax.axis_index("s")
        r0 = pl.multiple_of(sid * rows_per_tile, 8)

        pltpu.sync_copy(zcnt_hbm.at[pl.ds(r0, rows_per_tile)],
                        c_acc.at[pl.ds(r0, rows_per_tile)])
        pltpu.sync_copy(ones_hbm, ones_v)
        plsc.subcore_barrier()

        def outer(ob, carry):
            cb = pl.multiple_of(ob * IDXB, 8)
            pltpu.sync_copy(dst_hbm.at[c, sid, pl.ds(cb, IDXB)], idx_d)

            def step(j, cc):
                pltpu.sync_copy(ones_v, c_acc.at[idx_d.at[j]], add=True)
                return cc

            return lax.fori_loop(0, IDXB, step, carry, unroll=False)

        lax.fori_loop(0, n_chunks // IDXB, outer, 0, unroll=False)
        plsc.subcore_barrier()

        pltpu.sync_copy(c_acc.at[pl.ds(r0, rows_per_tile)],
                        cnt_hbm.at[c, pl.ds(r0, rows_per_tile)])

    return cntk


def _tc_fused(ssum, cnt, x, Wagg, Wself, b, apply_relu, blk=1000):
    n = x.shape[1]
    assert n % blk == 0

    def body(s_ref, c_ref, x_ref, wa_ref, ws_ref, b_ref, o_ref):
        cntcol = c_ref[0, :, 0:1]
        m = s_ref[0] / jnp.maximum(cntcol, 1.0)
        acc = jnp.dot(m, wa_ref[0], preferred_element_type=jnp.float32)
        acc = acc + jnp.dot(x_ref[0], ws_ref[0],
                            preferred_element_type=jnp.float32)
        acc = acc + b_ref[0]
        if apply_relu:
            acc = jnp.maximum(acc, 0.0)
        o_ref[0] = acc + x_ref[0]

    grid = (2, n // blk)
    return pl.pallas_call(
        body,
        grid=grid,
        in_specs=[
            pl.BlockSpec((1, blk, 128), lambda i, j: (i, j, 0)),
            pl.BlockSpec((1, blk, 128), lambda i, j: (i, j, 0)),
            pl.BlockSpec((1, blk, 128), lambda i, j: (i, j, 0)),
            pl.BlockSpec((1, 128, 128), lambda i, j: (i, 0, 0)),
            pl.BlockSpec((1, 128, 128), lambda i, j: (i, 0, 0)),
            pl.BlockSpec((1, 1, 128), lambda i, j: (i, 0, 0)),
        ],
        out_specs=pl.BlockSpec((1, blk, 128), lambda i, j: (i, j, 0)),
        out_shape=jax.ShapeDtypeStruct((2, n, 128), jnp.float32),
    )(ssum, cnt, x, Wagg, Wself, b.reshape(2, 1, 128))


def kernel(x_user, x_item, edge_index_u2i, edge_index_i2u, batch_user,
           batch_item, Wsu0, Wsi0, Wru0, Wri0, bu0, bi0,
           Wsu1, Wsi1, Wru1, Wri1, bu1, bi1):
    n_user, d = x_user.shape
    n_item = x_item.shape[0]
    e = edge_index_u2i.shape[1]
    assert n_user == n_item and d == 128
    n_dst = NS * 8 * (n_user // (NS * 8) + 1)

    n_chunks = IDXB * (-(-e // (NS * CHUNK * IDXB)))
    e_pad = NS * n_chunks * CHUNK
    pad = e_pad - e

    src0 = edge_index_u2i[0]
    dst0 = edge_index_u2i[1]
    src1 = edge_index_i2u[0] + n_user
    dst1 = edge_index_i2u[1]
    src = jnp.stack([
        jnp.pad(src0, (0, pad)),
        jnp.pad(src1, (0, pad), constant_values=n_user),
    ]).reshape(2, NS, n_chunks, CHUNK)
    dummy = n_dst - 1
    dst = jnp.stack([
        jnp.pad(dst0, (0, pad), constant_values=dummy),
        jnp.pad(dst1, (0, pad), constant_values=dummy),
    ]).reshape(2, NS, n_chunks, CHUNK)

    zrows = jnp.zeros((n_dst, 128), jnp.float32)
    ones = jnp.ones((CHUNK, 128), jnp.float32)

    agg = _make_sc_agg(n_dst, n_chunks)
    cntk = _make_sc_cnt(n_dst, n_chunks)

    Wagg0 = jnp.stack([Wru0, Wri0])
    Wself0 = jnp.stack([Wsi0, Wsu0])
    b0 = jnp.stack([bi0, bu0])
    Wagg1 = jnp.stack([Wru1, Wri1])
    Wself1 = jnp.stack([Wsi1, Wsu1])
    b1 = jnp.stack([bi1, bu1])

    cnt = cntk(dst, zrows, ones)[:, :n_user]

    table0 = jnp.concatenate([x_user, x_item], axis=0)
    ssum0 = agg(src, dst, table0, zrows)
    x_st = jnp.stack([x_item, x_user])
    x1 = _tc_fused(ssum0[:, :n_user], cnt, x_st,
                   Wagg0, Wself0, b0, apply_relu=True)

    table1 = jnp.concatenate([x1[1], x1[0]], axis=0)
    ssum1 = agg(src, dst, table1, zrows)
    x2 = _tc_fused(ssum1[:, :n_user], cnt, x1,
                   Wagg1, Wself1, b1, apply_relu=False)

    return x2[1], x2[0]


# speedup vs baseline: 1.8286x; 1.5722x over previous
import functools

import jax
import jax.numpy as jnp
from jax import lax
from jax.experimental import pallas as pl
from jax.experimental.pallas import tpu as pltpu
from jax.experimental.pallas import tpu_sc as plsc

NC = 2
NS = 16
CHUNK = 64
IDXB = 80


def _sc_mesh():
    return plsc.VectorSubcoreMesh(
        core_axis_name="c", subcore_axis_name="s",
        num_cores=NC, num_subcores=NS)


def _make_sc_agg(n_dst, n_chunks):
    assert n_dst % (NS * 8) == 0
    rows_per_tile = n_dst // NS

    @functools.partial(
        pl.kernel,
        out_type=jax.ShapeDtypeStruct((NC, n_dst, 128), jnp.float32),
        mesh=_sc_mesh(),
        scratch_types=[
            pltpu.VMEM((IDXB, CHUNK), jnp.int32),
            pltpu.VMEM((IDXB, CHUNK), jnp.int32),
            pltpu.VMEM((CHUNK, 128), jnp.float32),
            pltpu.VMEM((CHUNK, 128), jnp.float32),
            pltpu.VMEM_SHARED((n_dst, 128), jnp.float32),
            pltpu.SemaphoreType.DMA,
            pltpu.SemaphoreType.DMA,
        ],
    )
    def agg(src_hbm, dst_hbm, table_hbm, zrows_hbm, ssum_hbm,
            idx_s, idx_d, rows0, rows1, s_acc, sem0, sem1):
        c = lax.axis_index("c")
        sid = lax.axis_index("s")
        r0 = pl.multiple_of(sid * rows_per_tile, 8)

        pltpu.sync_copy(zrows_hbm.at[pl.ds(r0, rows_per_tile)],
                        s_acc.at[pl.ds(r0, rows_per_tile)])
        plsc.subcore_barrier()

        def start_g(j, buf, sem):
            pltpu.async_copy(table_hbm.at[idx_s.at[j]], buf, sem)

        def wait_g(buf, sem):
            pltpu.make_async_copy(zrows_hbm.at[pl.ds(0, CHUNK)], buf,
                                  sem).wait()

        def scat(j, buf):
            pltpu.sync_copy(buf, s_acc.at[idx_d.at[j]], add=True)

        def outer(ob, carry):
            cb = pl.multiple_of(ob * IDXB, 8)
            pltpu.sync_copy(src_hbm.at[c, sid, pl.ds(cb, IDXB)], idx_s)
            pltpu.sync_copy(dst_hbm.at[c, sid, pl.ds(cb, IDXB)], idx_d)
            start_g(0, rows0, sem0)

            def pair(j2, cc):
                j0 = 2 * j2
                wait_g(rows0, sem0)
                start_g(j0 + 1, rows1, sem1)
                scat(j0, rows0)
                wait_g(rows1, sem1)

                @pl.when(j0 + 2 < IDXB)
                def _():
                    start_g(j0 + 2, rows0, sem0)

                scat(j0 + 1, rows1)
                return cc

            return lax.fori_loop(0, IDXB // 2, pair, carry, unroll=False)

        lax.fori_loop(0, n_chunks // IDXB, outer, 0, unroll=False)
        plsc.subcore_barrier()

        pltpu.sync_copy(s_acc.at[pl.ds(r0, rows_per_tile)],
                        ssum_hbm.at[c, pl.ds(r0, rows_per_tile)])

    return agg


def _make_sc_cnt(n_dst, n_chunks):
    assert n_dst % (NS * 8) == 0
    rows_per_tile = n_dst // NS

    @functools.partial(
        pl.kernel,
        out_type=jax.ShapeDtypeStruct((NC, n_dst, 128), jnp.float32),
        mesh=_sc_mesh(),
        scratch_types=[
            pltpu.VMEM((IDXB, CHUNK), jnp.int32),
            pltpu.VMEM((CHUNK, 128), jnp.float32),
            pltpu.VMEM_SHARED((n_dst, 128), jnp.float32),
        ],
    )
    def cntk(dst_hbm, zcnt_hbm, ones_hbm, cnt_hbm, idx_d, ones_v, c_acc):
        c = lax.axis_index("c")
        sid = lax.axis_index("s")
        r0 = pl.multiple_of(sid * rows_per_tile, 8)

        pltpu.sync_copy(zcnt_hbm.at[pl.ds(r0, rows_per_tile)],
                        c_acc.at[pl.ds(r0, rows_per_tile)])
        pltpu.sync_copy(ones_hbm, ones_v)
        plsc.subcore_barrier()

        def outer(ob, carry):
            cb = pl.multiple_of(ob * IDXB, 8)
            pltpu.sync_copy(dst_hbm.at[c, sid, pl.ds(cb, IDXB)], idx_d)

            def step(j, cc):
                pltpu.sync_copy(ones_v, c_acc.at[idx_d.at[j]], add=True)
                return cc

            return lax.fori_loop(0, IDXB, step, carry, unroll=False)

        lax.fori_loop(0, n_chunks // IDXB, outer, 0, unroll=False)
        plsc.subcore_barrier()

        pltpu.sync_copy(c_acc.at[pl.ds(r0, rows_per_tile)],
                        cnt_hbm.at[c, pl.ds(r0, rows_per_tile)])

    return cntk


def _tc_fused(ssum, cnt, x, Wagg, Wself, b, apply_relu, blk=1000):
    n = x.shape[1]
    assert n % blk == 0

    def body(s_ref, c_ref, x_ref, wa_ref, ws_ref, b_ref, o_ref):
        cntcol = c_ref[0, :, 0:1]
        m = s_ref[0] / jnp.maximum(cntcol, 1.0)
        acc = jnp.dot(m, wa_ref[0], preferred_element_type=jnp.float32)
        acc = acc + jnp.dot(x_ref[0], ws_ref[0],
                            preferred_element_type=jnp.float32)
        acc = acc + b_ref[0]
        if apply_relu:
            acc = jnp.maximum(acc, 0.0)
        o_ref[0] = acc + x_ref[0]

    grid = (2, n // blk)
    return pl.pallas_call(
        body,
        grid=grid,
        in_specs=[
            pl.BlockSpec((1, blk, 128), lambda i, j: (i, j, 0)),
            pl.BlockSpec((1, blk, 128), lambda i, j: (i, j, 0)),
            pl.BlockSpec((1, blk, 128), lambda i, j: (i, j, 0)),
            pl.BlockSpec((1, 128, 128), lambda i, j: (i, 0, 0)),
            pl.BlockSpec((1, 128, 128), lambda i, j: (i, 0, 0)),
            pl.BlockSpec((1, 1, 128), lambda i, j: (i, 0, 0)),
        ],
        out_specs=pl.BlockSpec((1, blk, 128), lambda i, j: (i, j, 0)),
        out_shape=jax.ShapeDtypeStruct((2, n, 128), jnp.float32),
    )(ssum, cnt, x, Wagg, Wself, b.reshape(2, 1, 128))


def kernel(x_user, x_item, edge_index_u2i, edge_index_i2u, batch_user,
           batch_item, Wsu0, Wsi0, Wru0, Wri0, bu0, bi0,
           Wsu1, Wsi1, Wru1, Wri1, bu1, bi1):
    n_user, d = x_user.shape
    n_item = x_item.shape[0]
    e = edge_index_u2i.shape[1]
    assert n_user == n_item and d == 128
    n_dst = NS * 8 * (n_user // (NS * 8) + 1)

    n_chunks = IDXB * (-(-e // (NS * CHUNK * IDXB)))
    e_pad = NS * n_chunks * CHUNK
    pad = e_pad - e

    src0 = edge_index_u2i[0]
    dst0 = edge_index_u2i[1]
    src1 = edge_index_i2u[0] + n_user
    dst1 = edge_index_i2u[1]
    pad_src = jnp.arange(pad, dtype=jnp.int32) % n_user
    pad_dst = n_user + jnp.arange(pad, dtype=jnp.int32) % (n_dst - n_user)
    src = jnp.stack([
        jnp.concatenate([src0, pad_src]),
        jnp.concatenate([src1, pad_src + n_user]),
    ]).reshape(2, NS, n_chunks, CHUNK)
    dst = jnp.stack([
        jnp.concatenate([dst0, pad_dst]),
        jnp.concatenate([dst1, pad_dst]),
    ]).reshape(2, NS, n_chunks, CHUNK)

    zrows = jnp.zeros((n_dst, 128), jnp.float32)
    ones = jnp.ones((CHUNK, 128), jnp.float32)

    agg = _make_sc_agg(n_dst, n_chunks)
    cntk = _make_sc_cnt(n_dst, n_chunks)

    Wagg0 = jnp.stack([Wru0, Wri0])
    Wself0 = jnp.stack([Wsi0, Wsu0])
    b0 = jnp.stack([bi0, bu0])
    Wagg1 = jnp.stack([Wru1, Wri1])
    Wself1 = jnp.stack([Wsi1, Wsu1])
    b1 = jnp.stack([bi1, bu1])

    cnt = cntk(dst, zrows, ones)[:, :n_user]

    table0 = jnp.concatenate([x_user, x_item], axis=0)
    ssum0 = agg(src, dst, table0, zrows)
    x_st = jnp.stack([x_item, x_user])
    x1 = _tc_fused(ssum0[:, :n_user], cnt, x_st,
                   Wagg0, Wself0, b0, apply_relu=True)

    table1 = jnp.concatenate([x1[1], x1[0]], axis=0)
    ssum1 = agg(src, dst, table1, zrows)
    x2 = _tc_fused(ssum1[:, :n_user], cnt, x1,
                   Wagg1, Wself1, b1, apply_relu=False)

    return x2[1], x2[0]


# speedup vs baseline: 2.4046x; 1.3150x over previous
import functools

import jax
import jax.numpy as jnp
from jax import lax
from jax.experimental import pallas as pl
from jax.experimental.pallas import tpu as pltpu
from jax.experimental.pallas import tpu_sc as plsc

NC = 2
NS = 16
CHUNK = 64
IDXB = 48


def _sc_mesh():
    return plsc.VectorSubcoreMesh(
        core_axis_name="c", subcore_axis_name="s",
        num_cores=NC, num_subcores=NS)


def _make_sc_agg(n_dst, n_chunks):
    assert n_dst % (NS * 8) == 0
    rows_per_tile = n_dst // NS

    @functools.partial(
        pl.kernel,
        out_type=jax.ShapeDtypeStruct((NC, n_dst, 128), jnp.float32),
        mesh=_sc_mesh(),
        scratch_types=[
            pltpu.VMEM((IDXB, CHUNK), jnp.int32),
            pltpu.VMEM((IDXB, CHUNK), jnp.int32),
            pltpu.VMEM((CHUNK, 128), jnp.float32),
            pltpu.VMEM((CHUNK, 128), jnp.float32),
            pltpu.VMEM((CHUNK, 128), jnp.float32),
            pltpu.VMEM_SHARED((n_dst, 128), jnp.float32),
            pltpu.SemaphoreType.DMA,
            pltpu.SemaphoreType.DMA,
            pltpu.SemaphoreType.DMA,
        ],
    )
    def agg(src_hbm, dst_hbm, table_hbm, zrows_hbm, ssum_hbm,
            idx_s, idx_d, rows0, rows1, rows2, s_acc, sem0, sem1, sem2):
        c = lax.axis_index("c")
        sid = lax.axis_index("s")
        r0 = pl.multiple_of(sid * rows_per_tile, 8)

        pltpu.sync_copy(zrows_hbm.at[pl.ds(r0, rows_per_tile)],
                        s_acc.at[pl.ds(r0, rows_per_tile)])
        plsc.subcore_barrier()

        def start_g(j, buf, sem):
            pltpu.async_copy(table_hbm.at[idx_s.at[j]], buf, sem)

        def wait_g(buf, sem):
            pltpu.make_async_copy(zrows_hbm.at[pl.ds(0, CHUNK)], buf,
                                  sem).wait()

        def scat(j, buf):
            pltpu.sync_copy(buf, s_acc.at[idx_d.at[j]], add=True)

        def outer(ob, carry):
            cb = pl.multiple_of(ob * IDXB, 8)
            pltpu.sync_copy(src_hbm.at[c, sid, pl.ds(cb, IDXB)], idx_s)
            pltpu.sync_copy(dst_hbm.at[c, sid, pl.ds(cb, IDXB)], idx_d)
            start_g(0, rows0, sem0)
            start_g(1, rows1, sem1)

            def triple(j3, cc):
                j0 = 3 * j3
                wait_g(rows0, sem0)
                start_g(j0 + 2, rows2, sem2)
                scat(j0, rows0)
                wait_g(rows1, sem1)

                @pl.when(j0 + 3 < IDXB)
                def _():
                    start_g(j0 + 3, rows0, sem0)

                scat(j0 + 1, rows1)
                wait_g(rows2, sem2)

                @pl.when(j0 + 4 < IDXB)
                def _():
                    start_g(j0 + 4, rows1, sem1)

                scat(j0 + 2, rows2)
                return cc

            return lax.fori_loop(0, IDXB // 3, triple, carry, unroll=False)

        lax.fori_loop(0, n_chunks // IDXB, outer, 0, unroll=False)
        plsc.subcore_barrier()

        pltpu.sync_copy(s_acc.at[pl.ds(r0, rows_per_tile)],
                        ssum_hbm.at[c, pl.ds(r0, rows_per_tile)])

    return agg


def _make_sc_cnt(n_dst, n_chunks):
    assert n_dst % (NS * 8) == 0
    rows_per_tile = n_dst // NS

    @functools.partial(
        pl.kernel,
        out_type=jax.ShapeDtypeStruct((NC, n_dst, 128), jnp.float32),
        mesh=_sc_mesh(),
        scratch_types=[
            pltpu.VMEM((IDXB, CHUNK), jnp.int32),
            pltpu.VMEM((CHUNK, 128), jnp.float32),
            pltpu.VMEM_SHARED((n_dst, 128), jnp.float32),
        ],
    )
    def cntk(dst_hbm, zcnt_hbm, ones_hbm, cnt_hbm, idx_d, ones_v, c_acc):
        c = lax.axis_index("c")
        sid = lax.axis_index("s")
        r0 = pl.multiple_of(sid * rows_per_tile, 8)

        pltpu.sync_copy(zcnt_hbm.at[pl.ds(r0, rows_per_tile)],
                        c_acc.at[pl.ds(r0, rows_per_tile)])
        pltpu.sync_copy(ones_hbm, ones_v)
        plsc.subcore_barrier()

        def outer(ob, carry):
            cb = pl.multiple_of(ob * IDXB, 8)
            pltpu.sync_copy(dst_hbm.at[c, sid, pl.ds(cb, IDXB)], idx_d)

            def step(j, cc):
                pltpu.sync_copy(ones_v, c_acc.at[idx_d.at[j]], add=True)
                return cc

            return lax.fori_loop(0, IDXB, step, carry, unroll=False)

        lax.fori_loop(0, n_chunks // IDXB, outer, 0, unroll=False)
        plsc.subcore_barrier()

        pltpu.sync_copy(c_acc.at[pl.ds(r0, rows_per_tile)],
                        cnt_hbm.at[c, pl.ds(r0, rows_per_tile)])

    return cntk


def _tc_fused(ssum, cnt, x, Wagg, Wself, b, apply_relu, blk=1000):
    n = x.shape[1]
    assert n % blk == 0

    def body(s_ref, c_ref, x_ref, wa_ref, ws_ref, b_ref, o_ref):
        cntcol = c_ref[0, :, 0:1]
        m = s_ref[0] / jnp.maximum(cntcol, 1.0)
        acc = jnp.dot(m, wa_ref[0], preferred_element_type=jnp.float32)
        acc = acc + jnp.dot(x_ref[0], ws_ref[0],
                            preferred_element_type=jnp.float32)
        acc = acc + b_ref[0]
        if apply_relu:
            acc = jnp.maximum(acc, 0.0)
        o_ref[0] = acc + x_ref[0]

    grid = (2, n // blk)
    return pl.pallas_call(
        body,
        grid=grid,
        in_specs=[
            pl.BlockSpec((1, blk, 128), lambda i, j: (i, j, 0)),
            pl.BlockSpec((1, blk, 128), lambda i, j: (i, j, 0)),
            pl.BlockSpec((1, blk, 128), lambda i, j: (i, j, 0)),
            pl.BlockSpec((1, 128, 128), lambda i, j: (i, 0, 0)),
            pl.BlockSpec((1, 128, 128), lambda i, j: (i, 0, 0)),
            pl.BlockSpec((1, 1, 128), lambda i, j: (i, 0, 0)),
        ],
        out_specs=pl.BlockSpec((1, blk, 128), lambda i, j: (i, j, 0)),
        out_shape=jax.ShapeDtypeStruct((2, n, 128), jnp.float32),
    )(ssum, cnt, x, Wagg, Wself, b.reshape(2, 1, 128))


def kernel(x_user, x_item, edge_index_u2i, edge_index_i2u, batch_user,
           batch_item, Wsu0, Wsi0, Wru0, Wri0, bu0, bi0,
           Wsu1, Wsi1, Wru1, Wri1, bu1, bi1):
    n_user, d = x_user.shape
    n_item = x_item.shape[0]
    e = edge_index_u2i.shape[1]
    assert n_user == n_item and d == 128
    n_dst = NS * 8 * (n_user // (NS * 8) + 1)

    n_chunks = IDXB * (-(-e // (NS * CHUNK * IDXB)))
    e_pad = NS * n_chunks * CHUNK
    pad = e_pad - e

    src0 = edge_index_u2i[0]
    dst0 = edge_index_u2i[1]
    src1 = edge_index_i2u[0] + n_user
    dst1 = edge_index_i2u[1]
    pad_src = jnp.arange(pad, dtype=jnp.int32) % n_user
    pad_dst = n_user + jnp.arange(pad, dtype=jnp.int32) % (n_dst - n_user)
    src = jnp.stack([
        jnp.concatenate([src0, pad_src]),
        jnp.concatenate([src1, pad_src + n_user]),
    ]).reshape(2, NS, n_chunks, CHUNK)
    dst = jnp.stack([
        jnp.concatenate([dst0, pad_dst]),
        jnp.concatenate([dst1, pad_dst]),
    ]).reshape(2, NS, n_chunks, CHUNK)

    zrows = jnp.zeros((n_dst, 128), jnp.float32)
    ones = jnp.ones((CHUNK, 128), jnp.float32)

    agg = _make_sc_agg(n_dst, n_chunks)
    cntk = _make_sc_cnt(n_dst, n_chunks)

    Wagg0 = jnp.stack([Wru0, Wri0])
    Wself0 = jnp.stack([Wsi0, Wsu0])
    b0 = jnp.stack([bi0, bu0])
    Wagg1 = jnp.stack([Wru1, Wri1])
    Wself1 = jnp.stack([Wsi1, Wsu1])
    b1 = jnp.stack([bi1, bu1])

    cnt = cntk(dst, zrows, ones)[:, :n_user]

    table0 = jnp.concatenate([x_user, x_item], axis=0)
    ssum0 = agg(src, dst, table0, zrows)
    x_st = jnp.stack([x_item, x_user])
    x1 = _tc_fused(ssum0[:, :n_user], cnt, x_st,
                   Wagg0, Wself0, b0, apply_relu=True)

    table1 = jnp.concatenate([x1[1], x1[0]], axis=0)
    ssum1 = agg(src, dst, table1, zrows)
    x2 = _tc_fused(ssum1[:, :n_user], cnt, x1,
                   Wagg1, Wself1, b1, apply_relu=False)

    return x2[1], x2[0]


# speedup vs baseline: 2.4408x; 1.0151x over previous
import functools

import jax
import jax.numpy as jnp
from jax import lax
from jax.experimental import pallas as pl
from jax.experimental.pallas import tpu as pltpu
from jax.experimental.pallas import tpu_sc as plsc

NC = 2
NS = 16
CHUNK = 64
IDXB = 16


def _sc_mesh():
    return plsc.VectorSubcoreMesh(
        core_axis_name="c", subcore_axis_name="s",
        num_cores=NC, num_subcores=NS)


def _make_sc_agg(n_dst, n_chunks):
    assert n_dst % (NS * 8) == 0
    rows_per_tile = n_dst // NS

    @functools.partial(
        pl.kernel,
        out_type=jax.ShapeDtypeStruct((NC, n_dst, 128), jnp.float32),
        mesh=_sc_mesh(),
        scratch_types=[
            pltpu.VMEM((IDXB, CHUNK), jnp.int32),
            pltpu.VMEM((IDXB, CHUNK), jnp.int32),
            pltpu.VMEM((CHUNK, 128), jnp.float32),
            pltpu.VMEM((CHUNK, 128), jnp.float32),
            pltpu.VMEM((CHUNK, 128), jnp.float32),
            pltpu.VMEM((CHUNK, 128), jnp.float32),
            pltpu.VMEM_SHARED((n_dst, 128), jnp.float32),
            pltpu.SemaphoreType.DMA,
            pltpu.SemaphoreType.DMA,
            pltpu.SemaphoreType.DMA,
            pltpu.SemaphoreType.DMA,
        ],
    )
    def agg(src_hbm, dst_hbm, table_hbm, zrows_hbm, ssum_hbm,
            idx_s, idx_d, rows0, rows1, rows2, rows3, s_acc,
            sem0, sem1, sem2, sem3):
        c = lax.axis_index("c")
        sid = lax.axis_index("s")
        r0 = pl.multiple_of(sid * rows_per_tile, 8)

        pltpu.sync_copy(zrows_hbm.at[pl.ds(r0, rows_per_tile)],
                        s_acc.at[pl.ds(r0, rows_per_tile)])
        plsc.subcore_barrier()

        def start_g(j, buf, sem):
            pltpu.async_copy(table_hbm.at[idx_s.at[j]], buf, sem)

        def wait_g(buf, sem):
            pltpu.make_async_copy(zrows_hbm.at[pl.ds(0, CHUNK)], buf,
                                  sem).wait()

        def scat(j, buf):
            pltpu.sync_copy(buf, s_acc.at[idx_d.at[j]], add=True)

        def outer(ob, carry):
            cb = pl.multiple_of(ob * IDXB, 8)
            pltpu.sync_copy(src_hbm.at[c, sid, pl.ds(cb, IDXB)], idx_s)
            pltpu.sync_copy(dst_hbm.at[c, sid, pl.ds(cb, IDXB)], idx_d)
            start_g(0, rows0, sem0)
            start_g(1, rows1, sem1)
            start_g(2, rows2, sem2)

            def quad(j4, cc):
                j0 = 4 * j4
                wait_g(rows0, sem0)
                start_g(j0 + 3, rows3, sem3)
                scat(j0, rows0)
                wait_g(rows1, sem1)

                @pl.when(j0 + 4 < IDXB)
                def _():
                    start_g(j0 + 4, rows0, sem0)

                scat(j0 + 1, rows1)
                wait_g(rows2, sem2)

                @pl.when(j0 + 5 < IDXB)
                def _():
                    start_g(j0 + 5, rows1, sem1)

                scat(j0 + 2, rows2)
                wait_g(rows3, sem3)

                @pl.when(j0 + 6 < IDXB)
                def _():
                    start_g(j0 + 6, rows2, sem2)

                scat(j0 + 3, rows3)
                return cc

            return lax.fori_loop(0, IDXB // 4, quad, carry, unroll=False)

        lax.fori_loop(0, n_chunks // IDXB, outer, 0, unroll=False)
        plsc.subcore_barrier()

        pltpu.sync_copy(s_acc.at[pl.ds(r0, rows_per_tile)],
                        ssum_hbm.at[c, pl.ds(r0, rows_per_tile)])

    return agg


def _make_sc_cnt(n_dst, n_chunks):
    assert n_dst % (NS * 8) == 0
    rows_per_tile = n_dst // NS

    @functools.partial(
        pl.kernel,
        out_type=jax.ShapeDtypeStruct((NC, n_dst, 128), jnp.float32),
        mesh=_sc_mesh(),
        scratch_types=[
            pltpu.VMEM((IDXB, CHUNK), jnp.int32),
            pltpu.VMEM((CHUNK, 128), jnp.float32),
            pltpu.VMEM_SHARED((n_dst, 128), jnp.float32),
        ],
    )
    def cntk(dst_hbm, zcnt_hbm, ones_hbm, cnt_hbm, idx_d, ones_v, c_acc):
        c = lax.axis_index("c")
        sid = lax.axis_index("s")
        r0 = pl.multiple_of(sid * rows_per_tile, 8)

        pltpu.sync_copy(zcnt_hbm.at[pl.ds(r0, rows_per_tile)],
                        c_acc.at[pl.ds(r0, rows_per_tile)])
        pltpu.sync_copy(ones_hbm, ones_v)
        plsc.subcore_barrier()

        def outer(ob, carry):
            cb = pl.multiple_of(ob * IDXB, 8)
            pltpu.sync_copy(dst_hbm.at[c, sid, pl.ds(cb, IDXB)], idx_d)

            def step(j, cc):
                pltpu.sync_copy(ones_v, c_acc.at[idx_d.at[j]], add=True)
                return cc

            return lax.fori_loop(0, IDXB, step, carry, unroll=False)

        lax.fori_loop(0, n_chunks // IDXB, outer, 0, unroll=False)
        plsc.subcore_barrier()

        pltpu.sync_copy(c_acc.at[pl.ds(r0, rows_per_tile)],
                        cnt_hbm.at[c, pl.ds(r0, rows_per_tile)])

    return cntk


def _tc_fused(ssum, cnt, x, Wagg, Wself, b, apply_relu, blk=1000):
    n = x.shape[1]
    assert n % blk == 0

    def body(s_ref, c_ref, x_ref, wa_ref, ws_ref, b_ref, o_ref):
        cntcol = c_ref[0, :, 0:1]
        m = s_ref[0] / jnp.maximum(cntcol, 1.0)
        acc = jnp.dot(m, wa_ref[0], preferred_element_type=jnp.float32)
        acc = acc + jnp.dot(x_ref[0], ws_ref[0],
                            preferred_element_type=jnp.float32)
        acc = acc + b_ref[0]
        if apply_relu:
            acc = jnp.maximum(acc, 0.0)
        o_ref[0] = acc + x_ref[0]

    grid = (2, n // blk)
    return pl.pallas_call(
        body,
        grid=grid,
        in_specs=[
            pl.BlockSpec((1, blk, 128), lambda i, j: (i, j, 0)),
            pl.BlockSpec((1, blk, 128), lambda i, j: (i, j, 0)),
            pl.BlockSpec((1, blk, 128), lambda i, j: (i, j, 0)),
            pl.BlockSpec((1, 128, 128), lambda i, j: (i, 0, 0)),
            pl.BlockSpec((1, 128, 128), lambda i, j: (i, 0, 0)),
            pl.BlockSpec((1, 1, 128), lambda i, j: (i, 0, 0)),
        ],
        out_specs=pl.BlockSpec((1, blk, 128), lambda i, j: (i, j, 0)),
        out_shape=jax.ShapeDtypeStruct((2, n, 128), jnp.float32),
    )(ssum, cnt, x, Wagg, Wself, b.reshape(2, 1, 128))


def kernel(x_user, x_item, edge_index_u2i, edge_index_i2u, batch_user,
           batch_item, Wsu0, Wsi0, Wru0, Wri0, bu0, bi0,
           Wsu1, Wsi1, Wru1, Wri1, bu1, bi1):
    n_user, d = x_user.shape
    n_item = x_item.shape[0]
    e = edge_index_u2i.shape[1]
    assert n_user == n_item and d == 128
    n_dst = NS * 8 * (n_user // (NS * 8) + 1)

    n_chunks = IDXB * (-(-e // (NS * CHUNK * IDXB)))
    e_pad = NS * n_chunks * CHUNK
    pad = e_pad - e

    src0 = edge_index_u2i[0]
    dst0 = edge_index_u2i[1]
    src1 = edge_index_i2u[0] + n_user
    dst1 = edge_index_i2u[1]
    pad_src = jnp.arange(pad, dtype=jnp.int32) % n_user
    pad_dst = n_user + jnp.arange(pad, dtype=jnp.int32) % (n_dst - n_user)
    src = jnp.stack([
        jnp.concatenate([src0, pad_src]),
        jnp.concatenate([src1, pad_src + n_user]),
    ]).reshape(2, NS, n_chunks, CHUNK)
    dst = jnp.stack([
        jnp.concatenate([dst0, pad_dst]),
        jnp.concatenate([dst1, pad_dst]),
    ]).reshape(2, NS, n_chunks, CHUNK)

    zrows = jnp.zeros((n_dst, 128), jnp.float32)
    ones = jnp.ones((CHUNK, 128), jnp.float32)

    agg = _make_sc_agg(n_dst, n_chunks)
    cntk = _make_sc_cnt(n_dst, n_chunks)

    Wagg0 = jnp.stack([Wru0, Wri0])
    Wself0 = jnp.stack([Wsi0, Wsu0])
    b0 = jnp.stack([bi0, bu0])
    Wagg1 = jnp.stack([Wru1, Wri1])
    Wself1 = jnp.stack([Wsi1, Wsu1])
    b1 = jnp.stack([bi1, bu1])

    cnt = cntk(dst, zrows, ones)[:, :n_user]

    table0 = jnp.concatenate([x_user, x_item], axis=0)
    ssum0 = agg(src, dst, table0, zrows)
    x_st = jnp.stack([x_item, x_user])
    x1 = _tc_fused(ssum0[:, :n_user], cnt, x_st,
                   Wagg0, Wself0, b0, apply_relu=True)

    table1 = jnp.concatenate([x1[1], x1[0]], axis=0)
    ssum1 = agg(src, dst, table1, zrows)
    x2 = _tc_fused(ssum1[:, :n_user], cnt, x1,
                   Wagg1, Wself1, b1, apply_relu=False)

    return x2[1], x2[0]
